# baseline (speedup 1.0000x reference)
import jax
import jax.numpy as jnp
from jax.experimental import pallas as pl
from jax.experimental.pallas import tpu as pltpu

B, S, H, E, K = 4, 4096, 2048, 16, 2
AUX_COEF = 0.01
Z_COEF = 0.001
N = B * S
T = 1024
NBLK = N // (2 * T)


def _router_kernel(x0_ref, x1_ref, w_ref, rw_ref, se_ref, stats_ref):
    i = pl.program_id(0)

    w = w_ref[...]
    l0 = jax.lax.dot_general(
        x0_ref[...], w, dimension_numbers=(((1,), (1,)), ((), ())),
        preferred_element_type=jnp.float32)
    l1 = jax.lax.dot_general(
        x1_ref[...], w, dimension_numbers=(((1,), (1,)), ((), ())),
        preferred_element_type=jnp.float32)
    logits = jnp.concatenate([l0, l1], axis=0)

    m = jnp.max(logits, axis=-1, keepdims=True)
    ex = jnp.exp(logits - m)
    denom = jnp.sum(ex, axis=-1, keepdims=True)
    z = m + jnp.log(denom)

    idx = jax.lax.broadcasted_iota(jnp.int32, (2 * T, E), 1)
    a1 = jnp.min(jnp.where(logits == m, idx, E), axis=-1, keepdims=True)
    mask1 = idx == a1
    masked = jnp.where(mask1, -jnp.inf, logits)
    l2 = jnp.max(masked, axis=-1, keepdims=True)
    a2 = jnp.min(jnp.where(masked == l2, idx, E), axis=-1, keepdims=True)
    mask2 = idx == a2

    w1 = 1.0 / (1.0 + jnp.exp(l2 - m))
    rw_ref[...] = jnp.concatenate([w1, 1.0 - w1], axis=-1)
    se_ref[...] = jnp.concatenate([a1, a2], axis=-1)

    probs_sum = jnp.sum(ex * (1.0 / denom), axis=0, keepdims=True)
    counts = jnp.sum(mask1.astype(jnp.float32) + mask2.astype(jnp.float32),
                     axis=0, keepdims=True)
    zsq = jnp.sum(z * z, axis=0, keepdims=True)

    @pl.when(i == 0)
    def _init():
        stats_ref[...] = jnp.zeros_like(stats_ref)

    stats_ref[1:2, 0:E] += probs_sum
    stats_ref[2:3, 0:E] += counts
    stats_ref[3:4, 0:1] += zsq

    @pl.when(i == NBLK - 1)
    def _finish():
        ps = stats_ref[1:2, 0:E]
        cn = stats_ref[2:3, 0:E]
        zs = stats_ref[3:4, 0:1]
        aux = jnp.sum(cn * ps) * (float(E) / (float(N) * float(N)))
        loss = AUX_COEF * aux + Z_COEF * (zs / float(N))
        stats_ref[0:1, 0:1] = loss


def kernel(hidden_states, gate_w):
    x = hidden_states.reshape(N, H)
    rw, se, stats = pl.pallas_call(
        _router_kernel,
        grid=(NBLK,),
        in_specs=[
            pl.BlockSpec((T, H), lambda i: (2 * i, 0)),
            pl.BlockSpec((T, H), lambda i: (2 * i + 1, 0)),
            pl.BlockSpec((E, H), lambda i: (0, 0)),
        ],
        out_specs=[
            pl.BlockSpec((2 * T, K), lambda i: (i, 0)),
            pl.BlockSpec((2 * T, K), lambda i: (i, 0)),
            pl.BlockSpec((8, 128), lambda i: (0, 0)),
        ],
        out_shape=[
            jax.ShapeDtypeStruct((N, K), jnp.float32),
            jax.ShapeDtypeStruct((N, K), jnp.int32),
            jax.ShapeDtypeStruct((8, 128), jnp.float32),
        ],
    )(x, x, gate_w)
    return rw.reshape(B, S, K), se.reshape(B, S, K), stats[0, 0]


# speedup vs baseline: 1.4121x; 1.4121x over previous
import jax
import jax.numpy as jnp
from jax.experimental import pallas as pl
from jax.experimental.pallas import tpu as pltpu

B, S, H, E, K = 4, 4096, 2048, 16, 2
AUX_COEF = 0.01
Z_COEF = 0.001
N = B * S
T = 2048
NBLK = N // T


def _router_kernel(x_ref, w_ref, rw_ref, se_ref, stats_ref):
    i = pl.program_id(0)

    lt = jax.lax.dot_general(
        w_ref[...], x_ref[...],
        dimension_numbers=(((1,), (1,)), ((), ())),
        preferred_element_type=jnp.float32)

    m = jnp.max(lt, axis=0, keepdims=True)
    ex = jnp.exp(lt - m)
    denom = jnp.sum(ex, axis=0, keepdims=True)
    z = m + jnp.log(denom)

    sidx = jax.lax.broadcasted_iota(jnp.int32, (E, T), 0)
    a1 = jnp.min(jnp.where(lt == m, sidx, E), axis=0, keepdims=True)
    mask1 = sidx == a1
    masked = jnp.where(mask1, -jnp.inf, lt)
    l2 = jnp.max(masked, axis=0, keepdims=True)
    a2 = jnp.min(jnp.where(masked == l2, sidx, E), axis=0, keepdims=True)
    mask2 = sidx == a2

    w1 = 1.0 / (1.0 + jnp.exp(l2 - m))
    rw_ref[...] = jnp.concatenate([w1, 1.0 - w1], axis=0)
    se_ref[...] = jnp.concatenate([a1, a2], axis=0)

    probs_sum = jnp.sum(ex * (1.0 / denom), axis=1, keepdims=True)
    counts = jnp.sum(mask1.astype(jnp.float32) + mask2.astype(jnp.float32),
                     axis=1, keepdims=True)
    zsq = jnp.sum(z * z, axis=1, keepdims=True)

    @pl.when(i == 0)
    def _init():
        stats_ref[...] = jnp.zeros_like(stats_ref)

    stats_ref[0:E, 0:1] += probs_sum
    stats_ref[0:E, 1:2] += counts
    stats_ref[0:1, 2:3] += zsq

    @pl.when(i == NBLK - 1)
    def _finish():
        ps = stats_ref[0:E, 0:1]
        cn = stats_ref[0:E, 1:2]
        zs = stats_ref[0:1, 2:3]
        aux = jnp.sum(cn * ps) * (float(E) / (float(N) * float(N)))
        loss = AUX_COEF * aux + Z_COEF * (zs / float(N))
        stats_ref[0:1, 3:4] = loss


def kernel(hidden_states, gate_w):
    x = hidden_states.reshape(N, H)
    rw, se, stats = pl.pallas_call(
        _router_kernel,
        grid=(NBLK,),
        in_specs=[
            pl.BlockSpec((T, H), lambda i: (i, 0)),
            pl.BlockSpec((E, H), lambda i: (0, 0)),
        ],
        out_specs=[
            pl.BlockSpec((K, T), lambda i: (0, i)),
            pl.BlockSpec((K, T), lambda i: (0, i)),
            pl.BlockSpec((E, 128), lambda i: (0, 0)),
        ],
        out_shape=[
            jax.ShapeDtypeStruct((K, N), jnp.float32),
            jax.ShapeDtypeStruct((K, N), jnp.int32),
            jax.ShapeDtypeStruct((E, 128), jnp.float32),
        ],
    )(x, gate_w)
    routing_weights = rw.T.reshape(B, S, K)
    selected_experts = se.T.reshape(B, S, K)
    return routing_weights, selected_experts, stats[0, 3]
